# baseline (speedup 1.0000x reference)
import functools

import jax
import jax.numpy as jnp
from jax import lax
from jax.experimental import pallas as pl
from jax.experimental.pallas import tpu as pltpu
from jax.experimental.pallas import tpu_sc as plsc

N_USERS = 1000000
N_ITEMS = 1000000
D = 64
B = 16384

NC = 2
NS = 16
NW = NC * NS
BPW = B // NW
CHUNK = 128
NCHUNK = BPW // CHUNK


def _sc_body(u_hbm, i_hbm, U_hbm, V_hbm, ub_hbm, vb_hbm, gb_hbm, out_hbm,
             uidx_v, iidx_v, urows_v, vrows_v, ubr_v, vbr_v, out_v, gb_v, sem):
    wid = lax.axis_index("s") * NC + lax.axis_index("c")
    base = wid * BPW

    pltpu.sync_copy(u_hbm.at[pl.ds(wid * NCHUNK, NCHUNK)], uidx_v)
    pltpu.sync_copy(i_hbm.at[pl.ds(wid * NCHUNK, NCHUNK)], iidx_v)
    pltpu.sync_copy(gb_hbm, gb_v)

    copies = []
    for c in range(NCHUNK):
        rows = pl.ds(c * CHUNK, CHUNK)
        copies.append(pltpu.async_copy(U_hbm.at[uidx_v.at[c]], urows_v.at[rows], sem))
        copies.append(pltpu.async_copy(V_hbm.at[iidx_v.at[c]], vrows_v.at[rows], sem))
        copies.append(pltpu.async_copy(ub_hbm.at[uidx_v.at[c]], ubr_v.at[rows], sem))
        copies.append(pltpu.async_copy(vb_hbm.at[iidx_v.at[c]], vbr_v.at[rows], sem))
    for cp in copies:
        cp.wait()

    gb = gb_v[...]
    lanes0 = jnp.arange(16, dtype=jnp.int32)
    zeros = jnp.zeros((16,), jnp.int32)

    def j_body(j, carry):
        lanes = lanes0 + j * 16
        acc = ubr_v[pl.ds(j * 16, 16)] + vbr_v[pl.ds(j * 16, 16)] + gb
        for d in range(D):
            dsplat = jnp.full((16,), d, jnp.int32)
            acc = acc + (plsc.load_gather(urows_v, [lanes, dsplat])
                         * plsc.load_gather(vrows_v, [lanes, dsplat]))
        out_v[pl.ds(j * 16, 16)] = acc
        return carry

    lax.fori_loop(0, BPW // 16, j_body, 0)

    pltpu.sync_copy(out_v, out_hbm.at[pl.ds(base, BPW)])


@jax.jit
def _mf_scores(u2, i2, U, V, ub, vb, gb1):
    mesh = plsc.VectorSubcoreMesh(core_axis_name="c", subcore_axis_name="s")
    kern = pl.kernel(
        _sc_body,
        out_type=jax.ShapeDtypeStruct((B,), jnp.float32),
        mesh=mesh,
        compiler_params=pltpu.CompilerParams(
            needs_layout_passes=False, use_tc_tiling_on_sc=False),
        scratch_types=[
            pltpu.VMEM((NCHUNK, CHUNK), jnp.int32),
            pltpu.VMEM((NCHUNK, CHUNK), jnp.int32),
            pltpu.VMEM((BPW, D), jnp.float32),
            pltpu.VMEM((BPW, D), jnp.float32),
            pltpu.VMEM((BPW,), jnp.float32),
            pltpu.VMEM((BPW,), jnp.float32),
            pltpu.VMEM((BPW,), jnp.float32),
            pltpu.VMEM((16,), jnp.float32),
            pltpu.SemaphoreType.DMA,
        ],
    )
    return kern(u2, i2, U, V, ub, vb, gb1)


def kernel(u, i, U, V, ub, vb, gb):
    u2 = u.reshape(NW * NCHUNK, CHUNK)
    i2 = i.reshape(NW * NCHUNK, CHUNK)
    gb1 = jnp.broadcast_to(jnp.asarray(gb, jnp.float32), (16,))
    return _mf_scores(u2, i2, U, V, ub.reshape(N_USERS), vb.reshape(N_ITEMS), gb1)


# speedup vs baseline: 1.7954x; 1.7954x over previous
import functools

import jax
import jax.numpy as jnp
from jax import lax
from jax.experimental import pallas as pl
from jax.experimental.pallas import tpu as pltpu
from jax.experimental.pallas import tpu_sc as plsc

N_USERS = 1000000
N_ITEMS = 1000000
D = 64
B = 16384

NC = 2
NS = 16
NW = NC * NS
BPW = B // NW
CHUNK = 128
NCHUNK = BPW // CHUNK
G = 2
NG = BPW // G
SLAB = 128


def _sc_body(u_hbm, i_hbm, u3_hbm, i3_hbm, Ut_hbm, Vt_hbm, ub_hbm, vb_hbm,
             gb_hbm, out_hbm,
             uidx_v, iidx_v, uflat_v, iflat_v, uslab_v, vslab_v,
             ubr_v, vbr_v, out_v, gb_v, sem, gsem):
    wid = lax.axis_index("s") * NC + lax.axis_index("c")
    base = wid * BPW

    pltpu.sync_copy(u_hbm.at[pl.ds(wid * NCHUNK, NCHUNK)], uidx_v)
    pltpu.sync_copy(i_hbm.at[pl.ds(wid * NCHUNK, NCHUNK)], iidx_v)
    pltpu.sync_copy(u3_hbm.at[wid], uflat_v)
    pltpu.sync_copy(i3_hbm.at[wid], iflat_v)
    pltpu.sync_copy(gb_hbm, gb_v)

    bcopies = []
    for c in range(NCHUNK):
        rows = pl.ds(c * CHUNK, CHUNK)
        bcopies.append(pltpu.async_copy(ub_hbm.at[uidx_v.at[c]], ubr_v.at[rows], sem))
        bcopies.append(pltpu.async_copy(vb_hbm.at[iidx_v.at[c]], vbr_v.at[rows], sem))
    for cp in bcopies:
        cp.wait()

    gb = gb_v[...]
    lanes0 = jnp.arange(16, dtype=jnp.int32)
    mask128 = ~jnp.int32(127)

    def fire(g):
        buf = g & 1
        uvec = uflat_v[pl.ds((g // 8) * 16, 16)]
        ivec = iflat_v[pl.ds((g // 8) * 16, 16)]
        for l in range(G):
            lane = (g % 8) * G + l
            onel = jnp.where(lanes0 == lane, jnp.int32(1), jnp.int32(0))
            su = pl.multiple_of(jnp.sum(uvec * onel) & mask128, 128)
            si = pl.multiple_of(jnp.sum(ivec * onel) & mask128, 128)
            pltpu.async_copy(Ut_hbm.at[:, pl.ds(su, SLAB)], uslab_v.at[buf, l], gsem)
            pltpu.async_copy(Vt_hbm.at[:, pl.ds(si, SLAB)], vslab_v.at[buf, l], gsem)

    fire(jnp.int32(0))

    def group(g, acc):
        for _ in range(2 * G):
            pltpu.make_async_copy(
                Ut_hbm.at[:, pl.ds(0, SLAB)], uslab_v.at[0, 0], gsem).wait()

        @pl.when(g + 1 < NG)
        def _():
            fire(g + 1)

        buf = g & 1
        uvec = uflat_v[pl.ds((g // 8) * 16, 16)]
        ivec = iflat_v[pl.ds((g // 8) * 16, 16)]
        cu_all = uvec & 127
        cv_all = ivec & 127
        for l in range(G):
            lane = (g % 8) * G + l
            onel = jnp.where(lanes0 == lane, jnp.int32(1), jnp.int32(0))
            cu = jnp.full((16,), jnp.sum(cu_all * onel), jnp.int32)
            cv = jnp.full((16,), jnp.sum(cv_all * onel), jnp.int32)
            bufv = jnp.full((16,), buf, jnp.int32)
            lv = jnp.full((16,), l, jnp.int32)
            psum = jnp.zeros((16,), jnp.float32)
            for c in range(D // 16):
                dvec = lanes0 + c * 16
                psum = psum + (plsc.load_gather(uslab_v, [bufv, lv, dvec, cu])
                               * plsc.load_gather(vslab_v, [bufv, lv, dvec, cv]))
            s = jnp.sum(psum)
            acc = jnp.where(lanes0 == lane, s, acc)

        @pl.when(g % 8 == 7)
        def _():
            j = g // 8
            out_v[pl.ds(j * 16, 16)] = (acc + ubr_v[pl.ds(j * 16, 16)]
                                        + vbr_v[pl.ds(j * 16, 16)] + gb)
        return jnp.where(g % 8 == 7, jnp.zeros((16,), jnp.float32), acc)

    lax.fori_loop(0, NG, group, jnp.zeros((16,), jnp.float32))

    pltpu.sync_copy(out_v, out_hbm.at[pl.ds(base, BPW)])


@jax.jit
def _mf_scores(u2, i2, u3, i3, Ut, Vt, ubf, vbf, gb1):
    mesh = plsc.VectorSubcoreMesh(core_axis_name="c", subcore_axis_name="s")
    kern = pl.kernel(
        _sc_body,
        out_type=jax.ShapeDtypeStruct((B,), jnp.float32),
        mesh=mesh,
        compiler_params=pltpu.CompilerParams(
            needs_layout_passes=False, use_tc_tiling_on_sc=True),
        scratch_types=[
            pltpu.VMEM((NCHUNK, CHUNK), jnp.int32),
            pltpu.VMEM((NCHUNK, CHUNK), jnp.int32),
            pltpu.VMEM((BPW,), jnp.int32),
            pltpu.VMEM((BPW,), jnp.int32),
            pltpu.VMEM((2, G, D, SLAB), jnp.float32),
            pltpu.VMEM((2, G, D, SLAB), jnp.float32),
            pltpu.VMEM((BPW,), jnp.float32),
            pltpu.VMEM((BPW,), jnp.float32),
            pltpu.VMEM((BPW,), jnp.float32),
            pltpu.VMEM((16,), jnp.float32),
            pltpu.SemaphoreType.DMA,
            pltpu.SemaphoreType.DMA,
        ],
    )
    return kern(u2, i2, u3, i3, Ut, Vt, ubf, vbf, gb1)


def kernel(u, i, U, V, ub, vb, gb):
    u2 = u.reshape(NW * NCHUNK, CHUNK)
    i2 = i.reshape(NW * NCHUNK, CHUNK)
    u3 = u.reshape(NW, BPW)
    i3 = i.reshape(NW, BPW)
    gb1 = jnp.broadcast_to(jnp.asarray(gb, jnp.float32), (16,))
    return _mf_scores(u2, i2, u3, i3, U.T, V.T,
                      ub.reshape(N_USERS), vb.reshape(N_ITEMS), gb1)


# speedup vs baseline: 2.2263x; 1.2400x over previous
import functools

import jax
import jax.numpy as jnp
from jax import lax
from jax.experimental import pallas as pl
from jax.experimental.pallas import tpu as pltpu
from jax.experimental.pallas import tpu_sc as plsc

N_USERS = 1000000
N_ITEMS = 1000000
D = 64
B = 16384

NC = 2
NS = 16
NW = NC * NS
BPW = B // NW
CHUNK = 128
NCHUNK = BPW // CHUNK
R = 4
NSUP = BPW // R
SLAB = 128


def _sc_body(u_hbm, i_hbm, u3_hbm, i3_hbm, Ut_hbm, Vt_hbm, ub_hbm, vb_hbm,
             gb_hbm, out_hbm,
             uidx_v, iidx_v, uflat_v, iflat_v, uslab_v, vslab_v,
             ubr_v, vbr_v, out_v, gb_v, sem, s0, s1, s2, s3):
    wid = lax.axis_index("s") * NC + lax.axis_index("c")
    base = wid * BPW

    pltpu.sync_copy(u_hbm.at[pl.ds(wid * NCHUNK, NCHUNK)], uidx_v)
    pltpu.sync_copy(i_hbm.at[pl.ds(wid * NCHUNK, NCHUNK)], iidx_v)
    pltpu.sync_copy(u3_hbm.at[wid], uflat_v)
    pltpu.sync_copy(i3_hbm.at[wid], iflat_v)
    pltpu.sync_copy(gb_hbm, gb_v)

    bcopies = []
    for c in range(NCHUNK):
        rows = pl.ds(c * CHUNK, CHUNK)
        bcopies.append(pltpu.async_copy(ub_hbm.at[uidx_v.at[c]], ubr_v.at[rows], sem))
        bcopies.append(pltpu.async_copy(vb_hbm.at[iidx_v.at[c]], vbr_v.at[rows], sem))
    for cp in bcopies:
        cp.wait()

    gb = gb_v[...]
    lanes0 = jnp.arange(16, dtype=jnp.int32)
    mask128 = ~jnp.int32(127)
    sems = [s0, s1, s2, s3]

    def fire(l, j):
        uvec = uflat_v[pl.ds((l // 16) * 16, 16)]
        ivec = iflat_v[pl.ds((l // 16) * 16, 16)]
        onel = jnp.where(lanes0 == l % 16, jnp.int32(1), jnp.int32(0))
        su = pl.multiple_of(jnp.sum(uvec * onel) & mask128, 128)
        si = pl.multiple_of(jnp.sum(ivec * onel) & mask128, 128)
        pltpu.async_copy(Ut_hbm.at[:, pl.ds(su, SLAB)], uslab_v.at[j], sems[j])
        pltpu.async_copy(Vt_hbm.at[:, pl.ds(si, SLAB)], vslab_v.at[j], sems[j])

    for j in range(R):
        fire(jnp.int32(j), j)

    def super_iter(k, acc):
        for j in range(R):
            l = k * R + j
            pltpu.make_async_copy(
                Ut_hbm.at[:, pl.ds(0, SLAB)], uslab_v.at[j], sems[j]).wait()
            pltpu.make_async_copy(
                Vt_hbm.at[:, pl.ds(0, SLAB)], vslab_v.at[j], sems[j]).wait()

            uvec = uflat_v[pl.ds((l // 16) * 16, 16)]
            ivec = iflat_v[pl.ds((l // 16) * 16, 16)]
            lane = l % 16
            onel = jnp.where(lanes0 == lane, jnp.int32(1), jnp.int32(0))
            cu = jnp.full((16,), jnp.sum((uvec & 127) * onel), jnp.int32)
            cv = jnp.full((16,), jnp.sum((ivec & 127) * onel), jnp.int32)
            jv = jnp.full((16,), j, jnp.int32)
            psum = jnp.zeros((16,), jnp.float32)
            for c in range(D // 16):
                dvec = lanes0 + c * 16
                psum = psum + (plsc.load_gather(uslab_v, [jv, dvec, cu])
                               * plsc.load_gather(vslab_v, [jv, dvec, cv]))
            acc = jnp.where(lanes0 == lane, jnp.sum(psum), acc)

            @pl.when(k + 1 < NSUP)
            def _():
                fire(l + R, j)

        @pl.when((k % (16 // R)) == (16 // R) - 1)
        def _():
            jj = k // (16 // R)
            out_v[pl.ds(jj * 16, 16)] = (acc + ubr_v[pl.ds(jj * 16, 16)]
                                         + vbr_v[pl.ds(jj * 16, 16)] + gb)
        return jnp.where((k % (16 // R)) == (16 // R) - 1,
                         jnp.zeros((16,), jnp.float32), acc)

    lax.fori_loop(0, NSUP, super_iter, jnp.zeros((16,), jnp.float32))

    pltpu.sync_copy(out_v, out_hbm.at[pl.ds(base, BPW)])


@jax.jit
def _mf_scores(u2, i2, u3, i3, Ut, Vt, ubf, vbf, gb1):
    mesh = plsc.VectorSubcoreMesh(core_axis_name="c", subcore_axis_name="s")
    kern = pl.kernel(
        _sc_body,
        out_type=jax.ShapeDtypeStruct((B,), jnp.float32),
        mesh=mesh,
        compiler_params=pltpu.CompilerParams(
            needs_layout_passes=False, use_tc_tiling_on_sc=True),
        scratch_types=[
            pltpu.VMEM((NCHUNK, CHUNK), jnp.int32),
            pltpu.VMEM((NCHUNK, CHUNK), jnp.int32),
            pltpu.VMEM((BPW,), jnp.int32),
            pltpu.VMEM((BPW,), jnp.int32),
            pltpu.VMEM((R, D, SLAB), jnp.float32),
            pltpu.VMEM((R, D, SLAB), jnp.float32),
            pltpu.VMEM((BPW,), jnp.float32),
            pltpu.VMEM((BPW,), jnp.float32),
            pltpu.VMEM((BPW,), jnp.float32),
            pltpu.VMEM((16,), jnp.float32),
            pltpu.SemaphoreType.DMA,
            pltpu.SemaphoreType.DMA,
            pltpu.SemaphoreType.DMA,
            pltpu.SemaphoreType.DMA,
            pltpu.SemaphoreType.DMA,
        ],
    )
    return kern(u2, i2, u3, i3, Ut, Vt, ubf, vbf, gb1)


def kernel(u, i, U, V, ub, vb, gb):
    u2 = u.reshape(NW * NCHUNK, CHUNK)
    i2 = i.reshape(NW * NCHUNK, CHUNK)
    u3 = u.reshape(NW, BPW)
    i3 = i.reshape(NW, BPW)
    gb1 = jnp.broadcast_to(jnp.asarray(gb, jnp.float32), (16,))
    return _mf_scores(u2, i2, u3, i3, U.T, V.T,
                      ub.reshape(N_USERS), vb.reshape(N_ITEMS), gb1)


# speedup vs baseline: 2.4182x; 1.0862x over previous
import functools

import jax
import jax.numpy as jnp
from jax import lax
from jax.experimental import pallas as pl
from jax.experimental.pallas import tpu as pltpu
from jax.experimental.pallas import tpu_sc as plsc

N_USERS = 1000000
N_ITEMS = 1000000
D = 64
B = 16384

NC = 2
NS = 16
NW = NC * NS
BPW = B // NW
CHUNK = 128
NCHUNK = BPW // CHUNK
R = 6
NSUP = BPW // R
REM = BPW - NSUP * R
SLAB = 128


def _sc_body(u_hbm, i_hbm, u3_hbm, i3_hbm, Ut_hbm, Vt_hbm, ub_hbm, vb_hbm,
             gb_hbm, out_hbm,
             uidx_v, iidx_v, uflat_v, iflat_v, uslab_v, vslab_v,
             ubr_v, vbr_v, out_v, gb_v, sem, s0, s1, s2, s3, s4, s5):
    wid = lax.axis_index("s") * NC + lax.axis_index("c")
    base = wid * BPW

    pltpu.sync_copy(u_hbm.at[pl.ds(wid * NCHUNK, NCHUNK)], uidx_v)
    pltpu.sync_copy(i_hbm.at[pl.ds(wid * NCHUNK, NCHUNK)], iidx_v)
    pltpu.sync_copy(u3_hbm.at[wid], uflat_v)
    pltpu.sync_copy(i3_hbm.at[wid], iflat_v)
    pltpu.sync_copy(gb_hbm, gb_v)

    bcopies = []
    for c in range(NCHUNK):
        rows = pl.ds(c * CHUNK, CHUNK)
        bcopies.append(pltpu.async_copy(ub_hbm.at[uidx_v.at[c]], ubr_v.at[rows], sem))
        bcopies.append(pltpu.async_copy(vb_hbm.at[iidx_v.at[c]], vbr_v.at[rows], sem))
    for cp in bcopies:
        cp.wait()

    gb = gb_v[...]
    lanes0 = jnp.arange(16, dtype=jnp.int32)
    mask128 = ~jnp.int32(127)
    sems = [s0, s1, s2, s3, s4, s5]

    def fire(l, j):
        uvec = uflat_v[pl.ds((l // 16) * 16, 16)]
        ivec = iflat_v[pl.ds((l // 16) * 16, 16)]
        onel = jnp.where(lanes0 == l % 16, jnp.int32(1), jnp.int32(0))
        su = pl.multiple_of(jnp.sum(uvec * onel) & mask128, 128)
        si = pl.multiple_of(jnp.sum(ivec * onel) & mask128, 128)
        pltpu.async_copy(Ut_hbm.at[:, pl.ds(su, SLAB)], uslab_v.at[j], sems[j])
        pltpu.async_copy(Vt_hbm.at[:, pl.ds(si, SLAB)], vslab_v.at[j], sems[j])

    for j in range(R):
        fire(jnp.int32(j), j)

    def consume(l, j, acc, may_fire):
        pltpu.make_async_copy(
            Ut_hbm.at[:, pl.ds(0, SLAB)], uslab_v.at[j], sems[j]).wait()
        pltpu.make_async_copy(
            Vt_hbm.at[:, pl.ds(0, SLAB)], vslab_v.at[j], sems[j]).wait()

        uvec = uflat_v[pl.ds((l // 16) * 16, 16)]
        ivec = iflat_v[pl.ds((l // 16) * 16, 16)]
        lane = l % 16
        onel = jnp.where(lanes0 == lane, jnp.int32(1), jnp.int32(0))
        cu = jnp.full((16,), jnp.sum((uvec & 127) * onel), jnp.int32)
        cv = jnp.full((16,), jnp.sum((ivec & 127) * onel), jnp.int32)
        jv = jnp.full((16,), j, jnp.int32)
        psum = jnp.zeros((16,), jnp.float32)
        for c in range(D // 16):
            dvec = lanes0 + c * 16
            psum = psum + (plsc.load_gather(uslab_v, [jv, dvec, cu])
                           * plsc.load_gather(vslab_v, [jv, dvec, cv]))
        acc = jnp.where(lanes0 == lane, jnp.sum(psum), acc)

        if may_fire:
            @pl.when(l + R < BPW)
            def _():
                fire(l + R, j)

        is15 = lane == 15

        @pl.when(is15)
        def _():
            jj = l // 16
            out_v[pl.ds(jj * 16, 16)] = (acc + ubr_v[pl.ds(jj * 16, 16)]
                                         + vbr_v[pl.ds(jj * 16, 16)] + gb)
        return jnp.where(jnp.full((16,), is15), jnp.zeros((16,), jnp.float32), acc)

    def super_iter(k, acc):
        for j in range(R):
            acc = consume(k * R + j, j, acc, True)
        return acc

    acc = lax.fori_loop(0, NSUP, super_iter, jnp.zeros((16,), jnp.float32))
    for j in range(REM):
        acc = consume(jnp.int32(NSUP * R + j), j, acc, False)

    pltpu.sync_copy(out_v, out_hbm.at[pl.ds(base, BPW)])


@jax.jit
def _mf_scores(u2, i2, u3, i3, Ut, Vt, ubf, vbf, gb1):
    mesh = plsc.VectorSubcoreMesh(core_axis_name="c", subcore_axis_name="s")
    kern = pl.kernel(
        _sc_body,
        out_type=jax.ShapeDtypeStruct((B,), jnp.float32),
        mesh=mesh,
        compiler_params=pltpu.CompilerParams(
            needs_layout_passes=False, use_tc_tiling_on_sc=True),
        scratch_types=[
            pltpu.VMEM((NCHUNK, CHUNK), jnp.int32),
            pltpu.VMEM((NCHUNK, CHUNK), jnp.int32),
            pltpu.VMEM((BPW,), jnp.int32),
            pltpu.VMEM((BPW,), jnp.int32),
            pltpu.VMEM((R, D, SLAB), jnp.float32),
            pltpu.VMEM((R, D, SLAB), jnp.float32),
            pltpu.VMEM((BPW,), jnp.float32),
            pltpu.VMEM((BPW,), jnp.float32),
            pltpu.VMEM((BPW,), jnp.float32),
            pltpu.VMEM((16,), jnp.float32),
        ] + [pltpu.SemaphoreType.DMA] * (1 + R),
    )
    return kern(u2, i2, u3, i3, Ut, Vt, ubf, vbf, gb1)


def kernel(u, i, U, V, ub, vb, gb):
    u2 = u.reshape(NW * NCHUNK, CHUNK)
    i2 = i.reshape(NW * NCHUNK, CHUNK)
    u3 = u.reshape(NW, BPW)
    i3 = i.reshape(NW, BPW)
    gb1 = jnp.broadcast_to(jnp.asarray(gb, jnp.float32), (16,))
    return _mf_scores(u2, i2, u3, i3, U.T, V.T,
                      ub.reshape(N_USERS), vb.reshape(N_ITEMS), gb1)


# speedup vs baseline: 2.4270x; 1.0036x over previous
import functools

import jax
import jax.numpy as jnp
from jax import lax
from jax.experimental import pallas as pl
from jax.experimental.pallas import tpu as pltpu
from jax.experimental.pallas import tpu_sc as plsc

N_USERS = 1000000
N_ITEMS = 1000000
D = 64
B = 16384

NC = 2
NS = 16
NW = NC * NS
BPW = B // NW
CHUNK = 128
NCHUNK = BPW // CHUNK
R = 7
NSUP = BPW // R
REM = BPW - NSUP * R
SLAB = 128


def _sc_body(u_hbm, i_hbm, u3_hbm, i3_hbm, Ut_hbm, Vt_hbm, ub_hbm, vb_hbm,
             gb_hbm, out_hbm,
             uidx_v, iidx_v, uflat_v, iflat_v, uslab_v, vslab_v,
             ubr_v, vbr_v, out_v, gb_v, sem, s0, s1, s2, s3, s4, s5, s6):
    wid = lax.axis_index("s") * NC + lax.axis_index("c")
    base = wid * BPW

    pltpu.sync_copy(u_hbm.at[pl.ds(wid * NCHUNK, NCHUNK)], uidx_v)
    pltpu.sync_copy(i_hbm.at[pl.ds(wid * NCHUNK, NCHUNK)], iidx_v)
    pltpu.sync_copy(u3_hbm.at[wid], uflat_v)
    pltpu.sync_copy(i3_hbm.at[wid], iflat_v)
    pltpu.sync_copy(gb_hbm, gb_v)

    bcopies = []
    for c in range(NCHUNK):
        rows = pl.ds(c * CHUNK, CHUNK)
        bcopies.append(pltpu.async_copy(ub_hbm.at[uidx_v.at[c]], ubr_v.at[rows], sem))
        bcopies.append(pltpu.async_copy(vb_hbm.at[iidx_v.at[c]], vbr_v.at[rows], sem))
    for cp in bcopies:
        cp.wait()

    gb = gb_v[...]
    lanes0 = jnp.arange(16, dtype=jnp.int32)
    mask128 = ~jnp.int32(127)
    sems = [s0, s1, s2, s3, s4, s5, s6]
    zero16 = jnp.zeros((16,), jnp.int32)

    def fire(l, j):
        uvec = uflat_v[pl.ds((l // 16) * 16, 16)]
        ivec = iflat_v[pl.ds((l // 16) * 16, 16)]
        onel = jnp.where(lanes0 == l % 16, jnp.int32(1), jnp.int32(0))
        su = pl.multiple_of(jnp.sum(uvec * onel) & mask128, 128)
        si = pl.multiple_of(jnp.sum(ivec * onel) & mask128, 128)
        pltpu.async_copy(Ut_hbm.at[:, pl.ds(su, SLAB)], uslab_v.at[j], sems[j])
        pltpu.async_copy(Vt_hbm.at[:, pl.ds(si, SLAB)], vslab_v.at[j], sems[j])

    for j in range(R):
        fire(jnp.int32(j), j)

    def consume(l, j, acc, may_fire):
        pltpu.make_async_copy(
            Ut_hbm.at[:, pl.ds(0, SLAB)], uslab_v.at[j], sems[j]).wait()
        pltpu.make_async_copy(
            Vt_hbm.at[:, pl.ds(0, SLAB)], vslab_v.at[j], sems[j]).wait()

        uvec = uflat_v[pl.ds((l // 16) * 16, 16)]
        ivec = iflat_v[pl.ds((l // 16) * 16, 16)]
        lane = l % 16
        onel = jnp.where(lanes0 == lane, jnp.int32(1), jnp.int32(0))
        cu = jnp.full((16,), jnp.sum((uvec & 127) * onel), jnp.int32)
        cv = jnp.full((16,), jnp.sum((ivec & 127) * onel), jnp.int32)
        jv = jnp.full((16,), j, jnp.int32)
        psum = jnp.zeros((16,), jnp.float32)
        for c in range(D // 16):
            dvec = lanes0 + c * 16
            psum = psum + (plsc.load_gather(uslab_v, [jv, dvec, cu])
                           * plsc.load_gather(vslab_v, [jv, dvec, cv]))
        acc = jnp.where(lanes0 == lane, jnp.sum(psum), acc)

        if may_fire:
            @pl.when(l + R < BPW)
            def _():
                fire(l + R, j)

        is15 = lane == 15

        @pl.when(is15)
        def _():
            jj = l // 16
            out_v[pl.ds(jj * 16, 16)] = (acc + ubr_v[pl.ds(jj * 16, 16)]
                                         + vbr_v[pl.ds(jj * 16, 16)] + gb)
        return jnp.where(jnp.full((16,), is15), jnp.zeros((16,), jnp.float32), acc)

    def super_iter(k, acc):
        for j in range(R):
            acc = consume(k * R + j, j, acc, True)
        return acc

    acc = lax.fori_loop(0, NSUP, super_iter, jnp.zeros((16,), jnp.float32))
    for j in range(REM):
        acc = consume(jnp.int32(NSUP * R + j), j, acc, False)

    pltpu.sync_copy(out_v, out_hbm.at[pl.ds(base, BPW)])


@jax.jit
def _mf_scores(u2, i2, u3, i3, Ut, Vt, ubf, vbf, gb1):
    mesh = plsc.VectorSubcoreMesh(core_axis_name="c", subcore_axis_name="s")
    kern = pl.kernel(
        _sc_body,
        out_type=jax.ShapeDtypeStruct((B,), jnp.float32),
        mesh=mesh,
        compiler_params=pltpu.CompilerParams(
            needs_layout_passes=False, use_tc_tiling_on_sc=True),
        scratch_types=[
            pltpu.VMEM((NCHUNK, CHUNK), jnp.int32),
            pltpu.VMEM((NCHUNK, CHUNK), jnp.int32),
            pltpu.VMEM((BPW,), jnp.int32),
            pltpu.VMEM((BPW,), jnp.int32),
            pltpu.VMEM((R, D, SLAB), jnp.float32),
            pltpu.VMEM((R, D, SLAB), jnp.float32),
            pltpu.VMEM((BPW,), jnp.float32),
            pltpu.VMEM((BPW,), jnp.float32),
            pltpu.VMEM((BPW,), jnp.float32),
            pltpu.VMEM((16,), jnp.float32),
        ] + [pltpu.SemaphoreType.DMA] * (1 + R),
    )
    return kern(u2, i2, u3, i3, Ut, Vt, ubf, vbf, gb1)


def kernel(u, i, U, V, ub, vb, gb):
    u2 = u.reshape(NW * NCHUNK, CHUNK)
    i2 = i.reshape(NW * NCHUNK, CHUNK)
    u3 = u.reshape(NW, BPW)
    i3 = i.reshape(NW, BPW)
    gb1 = jnp.broadcast_to(jnp.asarray(gb, jnp.float32), (16,))
    return _mf_scores(u2, i2, u3, i3, U.T, V.T,
                      ub.reshape(N_USERS), vb.reshape(N_ITEMS), gb1)


# speedup vs baseline: 2.4272x; 1.0001x over previous
import functools

import jax
import jax.numpy as jnp
from jax import lax
from jax.experimental import pallas as pl
from jax.experimental.pallas import tpu as pltpu
from jax.experimental.pallas import tpu_sc as plsc

N_USERS = 1000000
N_ITEMS = 1000000
D = 64
B = 16384

NC = 2
NS = 16
NW = NC * NS
BPW = B // NW
CHUNK = 128
NCHUNK = BPW // CHUNK
R = 7
NSUP = BPW // R
REM = BPW - NSUP * R
SLAB = 128


def _sc_body(u3_hbm, i3_hbm, Ut_hbm, Vt_hbm, ub_hbm, vb_hbm,
             gb_hbm, out_hbm,
             uflat_v, iflat_v, uslab_v, vslab_v,
             ubr_v, vbr_v, out_v, gb_v, sem, s0, s1, s2, s3, s4, s5, s6):
    wid = lax.axis_index("s") * NC + lax.axis_index("c")
    base = wid * BPW

    pltpu.sync_copy(u3_hbm.at[wid], uflat_v)
    pltpu.sync_copy(i3_hbm.at[wid], iflat_v)
    pltpu.sync_copy(gb_hbm, gb_v)
    gb = gb_v[...]
    lanes0 = jnp.arange(16, dtype=jnp.int32)
    mask128 = ~jnp.int32(127)
    sems = [s0, s1, s2, s3, s4, s5, s6]
    zero16 = jnp.zeros((16,), jnp.int32)

    def fire(l, j):
        uvec = uflat_v[pl.ds((l // 16) * 16, 16)]
        ivec = iflat_v[pl.ds((l // 16) * 16, 16)]
        onel = jnp.where(lanes0 == l % 16, jnp.int32(1), jnp.int32(0))
        su = pl.multiple_of(jnp.sum(uvec * onel) & mask128, 128)
        si = pl.multiple_of(jnp.sum(ivec * onel) & mask128, 128)
        pltpu.async_copy(Ut_hbm.at[:, pl.ds(su, SLAB)], uslab_v.at[j], sems[j])
        pltpu.async_copy(Vt_hbm.at[:, pl.ds(si, SLAB)], vslab_v.at[j], sems[j])

    for j in range(R):
        fire(jnp.int32(j), j)

    bcopies = []
    for c in range(NCHUNK):
        rows = pl.ds(c * CHUNK, CHUNK)
        bcopies.append(pltpu.async_copy(ub_hbm.at[uflat_v.at[rows]], ubr_v.at[rows], sem))
        bcopies.append(pltpu.async_copy(vb_hbm.at[iflat_v.at[rows]], vbr_v.at[rows], sem))
    for cp in bcopies:
        cp.wait()

    def consume(l, j, acc, may_fire):
        pltpu.make_async_copy(
            Ut_hbm.at[:, pl.ds(0, SLAB)], uslab_v.at[j], sems[j]).wait()
        pltpu.make_async_copy(
            Vt_hbm.at[:, pl.ds(0, SLAB)], vslab_v.at[j], sems[j]).wait()

        uvec = uflat_v[pl.ds((l // 16) * 16, 16)]
        ivec = iflat_v[pl.ds((l // 16) * 16, 16)]
        lane = l % 16
        onel = jnp.where(lanes0 == lane, jnp.int32(1), jnp.int32(0))
        cu = jnp.full((16,), jnp.sum((uvec & 127) * onel), jnp.int32)
        cv = jnp.full((16,), jnp.sum((ivec & 127) * onel), jnp.int32)
        jv = jnp.full((16,), j, jnp.int32)
        psum = jnp.zeros((16,), jnp.float32)
        for c in range(D // 16):
            dvec = lanes0 + c * 16
            psum = psum + (plsc.load_gather(uslab_v, [jv, dvec, cu])
                           * plsc.load_gather(vslab_v, [jv, dvec, cv]))
        acc = jnp.where(lanes0 == lane, jnp.sum(psum), acc)

        if may_fire:
            @pl.when(l + R < BPW)
            def _():
                fire(l + R, j)

        is15 = lane == 15

        @pl.when(is15)
        def _():
            jj = l // 16
            out_v[pl.ds(jj * 16, 16)] = (acc + ubr_v[pl.ds(jj * 16, 16)]
                                         + vbr_v[pl.ds(jj * 16, 16)] + gb)
        return jnp.where(jnp.full((16,), is15), jnp.zeros((16,), jnp.float32), acc)

    def super_iter(k, acc):
        for j in range(R):
            acc = consume(k * R + j, j, acc, True)
        return acc

    acc = lax.fori_loop(0, NSUP, super_iter, jnp.zeros((16,), jnp.float32))
    for j in range(REM):
        acc = consume(jnp.int32(NSUP * R + j), j, acc, False)

    pltpu.sync_copy(out_v, out_hbm.at[pl.ds(base, BPW)])


@jax.jit
def _mf_scores(u3, i3, Ut, Vt, ubf, vbf, gb1):
    mesh = plsc.VectorSubcoreMesh(core_axis_name="c", subcore_axis_name="s")
    kern = pl.kernel(
        _sc_body,
        out_type=jax.ShapeDtypeStruct((B,), jnp.float32),
        mesh=mesh,
        compiler_params=pltpu.CompilerParams(
            needs_layout_passes=False, use_tc_tiling_on_sc=True),
        scratch_types=[
            pltpu.VMEM((BPW,), jnp.int32),
            pltpu.VMEM((BPW,), jnp.int32),
            pltpu.VMEM((R, D, SLAB), jnp.float32),
            pltpu.VMEM((R, D, SLAB), jnp.float32),
            pltpu.VMEM((BPW,), jnp.float32),
            pltpu.VMEM((BPW,), jnp.float32),
            pltpu.VMEM((BPW,), jnp.float32),
            pltpu.VMEM((16,), jnp.float32),
        ] + [pltpu.SemaphoreType.DMA] * (1 + R),
    )
    return kern(u3, i3, Ut, Vt, ubf, vbf, gb1)


def kernel(u, i, U, V, ub, vb, gb):
    u3 = u.reshape(NW, BPW)
    i3 = i.reshape(NW, BPW)
    gb1 = jnp.broadcast_to(jnp.asarray(gb, jnp.float32), (16,))
    return _mf_scores(u3, i3, U.T, V.T,
                      ub.reshape(N_USERS), vb.reshape(N_ITEMS), gb1)


# speedup vs baseline: 2.4280x; 1.0004x over previous
import jax
import jax.numpy as jnp
from jax import lax
from jax.experimental import pallas as pl
from jax.experimental.pallas import tpu as pltpu
from jax.experimental.pallas import tpu_sc as plsc

N_USERS = 1000000
N_ITEMS = 1000000
D = 64
B = 16384

NC = 2
NS = 16
NW = NC * NS
BPW = B // NW
CHUNK = 128
NCHUNK = BPW // CHUNK
R = 7
NSUP = BPW // R
REM = BPW - NSUP * R
SLAB = 128


def _sc_body(u3_hbm, i3_hbm, Ut_hbm, Vt_hbm, ub_hbm, vb_hbm,
             gb_hbm, out_hbm,
             uflat_v, iflat_v, uslab_v, vslab_v,
             ubr_v, vbr_v, out_v, gb_v, sem, s0, s1, s2, s3, s4, s5, s6):
    wid = lax.axis_index("s") * NC + lax.axis_index("c")
    base = wid * BPW

    pltpu.sync_copy(u3_hbm.at[wid], uflat_v)
    pltpu.sync_copy(i3_hbm.at[wid], iflat_v)
    pltpu.sync_copy(gb_hbm, gb_v)
    gb = gb_v[...]
    lanes0 = jnp.arange(16, dtype=jnp.int32)
    mask128 = ~jnp.int32(127)
    sems = [s0, s1, s2, s3, s4, s5, s6]

    def fire(l, j):
        uvec = uflat_v[pl.ds((l // 16) * 16, 16)]
        ivec = iflat_v[pl.ds((l // 16) * 16, 16)]
        onel = jnp.where(lanes0 == l % 16, jnp.int32(1), jnp.int32(0))
        su = pl.multiple_of(jnp.sum(uvec * onel) & mask128, 128)
        si = pl.multiple_of(jnp.sum(ivec * onel) & mask128, 128)
        pltpu.async_copy(Ut_hbm.at[:, pl.ds(su, SLAB)], uslab_v.at[j], sems[j])
        pltpu.async_copy(Vt_hbm.at[:, pl.ds(si, SLAB)], vslab_v.at[j], sems[j])

    for j in range(R):
        fire(jnp.int32(j), j)

    bcopies = []
    for c in range(NCHUNK):
        rows = pl.ds(c * CHUNK, CHUNK)
        bcopies.append(pltpu.async_copy(ub_hbm.at[uflat_v.at[rows]], ubr_v.at[rows], sem))
        bcopies.append(pltpu.async_copy(vb_hbm.at[iflat_v.at[rows]], vbr_v.at[rows], sem))
    for cp in bcopies:
        cp.wait()

    def consume(l, j, acc, may_fire):
        pltpu.make_async_copy(
            Ut_hbm.at[:, pl.ds(0, SLAB)], uslab_v.at[j], sems[j]).wait()
        pltpu.make_async_copy(
            Vt_hbm.at[:, pl.ds(0, SLAB)], vslab_v.at[j], sems[j]).wait()

        uvec = uflat_v[pl.ds((l // 16) * 16, 16)]
        ivec = iflat_v[pl.ds((l // 16) * 16, 16)]
        lane = l % 16
        onel = jnp.where(lanes0 == lane, jnp.int32(1), jnp.int32(0))
        cu = jnp.full((16,), jnp.sum((uvec & 127) * onel), jnp.int32)
        cv = jnp.full((16,), jnp.sum((ivec & 127) * onel), jnp.int32)
        jv = jnp.full((16,), j, jnp.int32)
        psum = jnp.zeros((16,), jnp.float32)
        for c in range(D // 16):
            dvec = lanes0 + c * 16
            psum = psum + (plsc.load_gather(uslab_v, [jv, dvec, cu])
                           * plsc.load_gather(vslab_v, [jv, dvec, cv]))
        acc = jnp.where(lanes0 == lane, jnp.sum(psum), acc)

        if may_fire:
            @pl.when(l + R < BPW)
            def _():
                fire(l + R, j)

        is15 = lane == 15

        @pl.when(is15)
        def _():
            jj = l // 16
            out_v[pl.ds(jj * 16, 16)] = (acc + ubr_v[pl.ds(jj * 16, 16)]
                                         + vbr_v[pl.ds(jj * 16, 16)] + gb)
        return jnp.where(jnp.full((16,), is15), jnp.zeros((16,), jnp.float32), acc)

    def super_iter(k, acc):
        for j in range(R):
            acc = consume(k * R + j, j, acc, True)
        return acc

    acc = lax.fori_loop(0, NSUP, super_iter, jnp.zeros((16,), jnp.float32))
    for j in range(REM):
        acc = consume(jnp.int32(NSUP * R + j), j, acc, False)

    pltpu.sync_copy(out_v, out_hbm.at[pl.ds(base, BPW)])


@jax.jit
def _mf_scores(u3, i3, Ut, Vt, ubf, vbf, gb1):
    mesh = plsc.VectorSubcoreMesh(core_axis_name="c", subcore_axis_name="s")
    kern = pl.kernel(
        _sc_body,
        out_type=jax.ShapeDtypeStruct((B,), jnp.float32),
        mesh=mesh,
        compiler_params=pltpu.CompilerParams(
            needs_layout_passes=False, use_tc_tiling_on_sc=True),
        scratch_types=[
            pltpu.VMEM((BPW,), jnp.int32),
            pltpu.VMEM((BPW,), jnp.int32),
            pltpu.VMEM((R, D, SLAB), jnp.float32),
            pltpu.VMEM((R, D, SLAB), jnp.float32),
            pltpu.VMEM((BPW,), jnp.float32),
            pltpu.VMEM((BPW,), jnp.float32),
            pltpu.VMEM((BPW,), jnp.float32),
            pltpu.VMEM((16,), jnp.float32),
        ] + [pltpu.SemaphoreType.DMA] * (1 + R),
    )
    return kern(u3, i3, Ut, Vt, ubf, vbf, gb1)


def kernel(u, i, U, V, ub, vb, gb):
    u3 = u.reshape(NW, BPW)
    i3 = i.reshape(NW, BPW)
    gb1 = jnp.broadcast_to(jnp.asarray(gb, jnp.float32), (16,))
    return _mf_scores(u3, i3, U.T, V.T,
                      ub.reshape(N_USERS), vb.reshape(N_ITEMS), gb1)
